# baseline (device time: 101958 ns/iter reference)
import jax
import jax.numpy as jnp
from jax import lax
from jax.experimental import pallas as pl
from jax.experimental.pallas import tpu as pltpu

N_DEV = 8


def kernel(x, w_mat):
    m_full, k_per = x.shape
    k_per2, n = w_mat.shape
    assert k_per == k_per2
    m_per = m_full // N_DEV

    def body(x_ref, w_ref, out_ref, comm_ref, send_sems, recv_sems):
        my = lax.axis_index("i")
        left = lax.rem(my + (N_DEV - 1), N_DEV)
        right = lax.rem(my + 1, N_DEV)

        barrier_sem = pltpu.get_barrier_semaphore()
        for nbr in (left, right):
            pl.semaphore_signal(
                barrier_sem, inc=1,
                device_id=(nbr,), device_id_type=pl.DeviceIdType.MESH,
            )
        pl.semaphore_wait(barrier_sem, 2)

        def block(c):
            xb = x_ref[pl.ds(c * m_per, m_per), :]
            return jnp.dot(xb, w_ref[...], preferred_element_type=jnp.float32)

        c0 = lax.rem(my + (N_DEV - 1), N_DEV)
        comm_ref[0] = block(c0).astype(jnp.bfloat16)

        for h in range(N_DEV - 1):
            rdma = pltpu.make_async_remote_copy(
                src_ref=comm_ref.at[h],
                dst_ref=comm_ref.at[h + 1],
                send_sem=send_sems.at[h],
                recv_sem=recv_sems.at[h],
                device_id=(right,),
                device_id_type=pl.DeviceIdType.MESH,
            )
            rdma.start()
            rdma.wait()

            c = lax.rem(my + (2 * N_DEV - 2 - h), N_DEV)
            acc = comm_ref[h + 1].astype(jnp.float32) + block(c)
            if h < N_DEV - 2:
                comm_ref[h + 1] = acc.astype(jnp.bfloat16)
            else:
                out_ref[...] = jnp.maximum(acc, 0.0)

    return pl.pallas_call(
        body,
        out_shape=jax.ShapeDtypeStruct((m_per, n), jnp.float32),
        in_specs=[
            pl.BlockSpec(memory_space=pltpu.VMEM),
            pl.BlockSpec(memory_space=pltpu.VMEM),
        ],
        out_specs=pl.BlockSpec(memory_space=pltpu.VMEM),
        scratch_shapes=[
            pltpu.VMEM((N_DEV, m_per, n), jnp.bfloat16),
            pltpu.SemaphoreType.DMA((N_DEV - 1,)),
            pltpu.SemaphoreType.DMA((N_DEV - 1,)),
        ],
        compiler_params=pltpu.CompilerParams(collective_id=0),
    )(x, w_mat)


# device time: 51062 ns/iter; 1.9967x vs baseline; 1.9967x over previous
import jax
import jax.numpy as jnp
from jax import lax
from jax.experimental import pallas as pl
from jax.experimental.pallas import tpu as pltpu

N_DEV = 8
S = 2


def kernel(x, w_mat):
    m_full, k_per = x.shape
    k_per2, n = w_mat.shape
    assert k_per == k_per2
    m_per = m_full // N_DEV
    stripe = n // (2 * S)

    def body(x_ref, w_ref, out_ref,
             comm_cw, comm_ccw, send_cw, recv_cw, send_ccw, recv_ccw):
        my = lax.axis_index("i")
        left = lax.rem(my + (N_DEV - 1), N_DEV)
        right = lax.rem(my + 1, N_DEV)

        barrier_sem = pltpu.get_barrier_semaphore()
        for nbr in (left, right):
            pl.semaphore_signal(
                barrier_sem, inc=1,
                device_id=(nbr,), device_id_type=pl.DeviceIdType.MESH,
            )
        pl.semaphore_wait(barrier_sem, 2)

        def stripe_dot(c, st):
            xb = x_ref[pl.ds(c * m_per, m_per), :]
            wb = w_ref[:, st * stripe:(st + 1) * stripe]
            return jnp.dot(xb, wb, preferred_element_type=jnp.float32)

        def c_cw(h):
            return lax.rem(my + (2 * N_DEV - 2 - h), N_DEV)

        def c_ccw(h):
            return lax.rem(my + 2 + h, N_DEV)

        def mk(h, s, cw):
            comm = comm_cw if cw else comm_ccw
            return pltpu.make_async_remote_copy(
                src_ref=comm.at[s, h],
                dst_ref=comm.at[s, h + 1],
                send_sem=(send_cw if cw else send_ccw).at[s, h],
                recv_sem=(recv_cw if cw else recv_ccw).at[s, h],
                device_id=(right if cw else left,),
                device_id_type=pl.DeviceIdType.MESH,
            )

        descrs = {}
        for s in range(S):
            comm_cw[s, 0] = stripe_dot(
                lax.rem(my + (N_DEV - 1), N_DEV), s).astype(jnp.bfloat16)
            d = mk(0, s, True)
            d.start()
            descrs[(0, s, True)] = d
            comm_ccw[s, 0] = stripe_dot(
                lax.rem(my + 1, N_DEV), S + s).astype(jnp.bfloat16)
            d = mk(0, s, False)
            d.start()
            descrs[(0, s, False)] = d

        for h in range(N_DEV - 1):
            for s in range(S):
                for cw in (True, False):
                    st = s if cw else S + s
                    c = c_cw(h) if cw else c_ccw(h)
                    bl = stripe_dot(c, st)
                    d = descrs[(h, s, cw)]
                    d.wait_recv()
                    comm = comm_cw if cw else comm_ccw
                    acc = comm[s, h + 1].astype(jnp.float32) + bl
                    if h < N_DEV - 2:
                        comm[s, h + 1] = acc.astype(jnp.bfloat16)
                        d2 = mk(h + 1, s, cw)
                        d2.start()
                        descrs[(h + 1, s, cw)] = d2
                    else:
                        out_ref[:, st * stripe:(st + 1) * stripe] = (
                            jnp.maximum(acc, 0.0))

        for d in descrs.values():
            d.wait_send()

    return pl.pallas_call(
        body,
        out_shape=jax.ShapeDtypeStruct((m_per, n), jnp.float32),
        in_specs=[
            pl.BlockSpec(memory_space=pltpu.VMEM),
            pl.BlockSpec(memory_space=pltpu.VMEM),
        ],
        out_specs=pl.BlockSpec(memory_space=pltpu.VMEM),
        scratch_shapes=[
            pltpu.VMEM((S, N_DEV, m_per, stripe), jnp.bfloat16),
            pltpu.VMEM((S, N_DEV, m_per, stripe), jnp.bfloat16),
            pltpu.SemaphoreType.DMA((S, N_DEV - 1)),
            pltpu.SemaphoreType.DMA((S, N_DEV - 1)),
            pltpu.SemaphoreType.DMA((S, N_DEV - 1)),
            pltpu.SemaphoreType.DMA((S, N_DEV - 1)),
        ],
        compiler_params=pltpu.CompilerParams(collective_id=0),
    )(x, w_mat)


# device time: 50148 ns/iter; 2.0331x vs baseline; 1.0182x over previous
import jax
import jax.numpy as jnp
from jax import lax
from jax.experimental import pallas as pl
from jax.experimental.pallas import tpu as pltpu

N_DEV = 8
S = 2


def kernel(x, w_mat):
    m_full, k_per = x.shape
    k_per2, n = w_mat.shape
    assert k_per == k_per2
    m_per = m_full // N_DEV
    stripe = n // (2 * S)

    def body(x_ref, w_ref, out_ref,
             comm_cw, comm_ccw, send_cw, recv_cw, send_ccw, recv_ccw):
        my = lax.axis_index("i")

        def ring2log(q):
            return jnp.where(q < 4, q, 11 - q)

        p = ring2log(my)
        left = ring2log(lax.rem(p + (N_DEV - 1), N_DEV))
        right = ring2log(lax.rem(p + 1, N_DEV))

        barrier_sem = pltpu.get_barrier_semaphore()
        for nbr in (left, right):
            pl.semaphore_signal(
                barrier_sem, inc=1,
                device_id=(nbr,), device_id_type=pl.DeviceIdType.MESH,
            )
        pl.semaphore_wait(barrier_sem, 2)

        def stripe_dot(c, st):
            xb = x_ref[pl.ds(c * m_per, m_per), :]
            wb = w_ref[:, st * stripe:(st + 1) * stripe]
            return jnp.dot(xb, wb, preferred_element_type=jnp.float32)

        def c_cw(h):
            return ring2log(lax.rem(p + (2 * N_DEV - 2 - h), N_DEV))

        def c_ccw(h):
            return ring2log(lax.rem(p + 2 + h, N_DEV))

        def mk(h, s, cw):
            comm = comm_cw if cw else comm_ccw
            return pltpu.make_async_remote_copy(
                src_ref=comm.at[s, h],
                dst_ref=comm.at[s, h + 1],
                send_sem=(send_cw if cw else send_ccw).at[s, h],
                recv_sem=(recv_cw if cw else recv_ccw).at[s, h],
                device_id=(right if cw else left,),
                device_id_type=pl.DeviceIdType.MESH,
            )

        descrs = {}
        for s in range(S):
            comm_cw[s, 0] = stripe_dot(left, s).astype(jnp.bfloat16)
            d = mk(0, s, True)
            d.start()
            descrs[(0, s, True)] = d
            comm_ccw[s, 0] = stripe_dot(right, S + s).astype(jnp.bfloat16)
            d = mk(0, s, False)
            d.start()
            descrs[(0, s, False)] = d

        for h in range(N_DEV - 1):
            for s in range(S):
                for cw in (True, False):
                    st = s if cw else S + s
                    c = c_cw(h) if cw else c_ccw(h)
                    bl = stripe_dot(c, st)
                    d = descrs[(h, s, cw)]
                    d.wait_recv()
                    comm = comm_cw if cw else comm_ccw
                    acc = comm[s, h + 1].astype(jnp.float32) + bl
                    if h < N_DEV - 2:
                        comm[s, h + 1] = acc.astype(jnp.bfloat16)
                        d2 = mk(h + 1, s, cw)
                        d2.start()
                        descrs[(h + 1, s, cw)] = d2
                    else:
                        out_ref[:, st * stripe:(st + 1) * stripe] = (
                            jnp.maximum(acc, 0.0))

        for d in descrs.values():
            d.wait_send()

    return pl.pallas_call(
        body,
        out_shape=jax.ShapeDtypeStruct((m_per, n), jnp.float32),
        in_specs=[
            pl.BlockSpec(memory_space=pltpu.VMEM),
            pl.BlockSpec(memory_space=pltpu.VMEM),
        ],
        out_specs=pl.BlockSpec(memory_space=pltpu.VMEM),
        scratch_shapes=[
            pltpu.VMEM((S, N_DEV, m_per, stripe), jnp.bfloat16),
            pltpu.VMEM((S, N_DEV, m_per, stripe), jnp.bfloat16),
            pltpu.SemaphoreType.DMA((S, N_DEV - 1)),
            pltpu.SemaphoreType.DMA((S, N_DEV - 1)),
            pltpu.SemaphoreType.DMA((S, N_DEV - 1)),
            pltpu.SemaphoreType.DMA((S, N_DEV - 1)),
        ],
        compiler_params=pltpu.CompilerParams(collective_id=0),
    )(x, w_mat)
